# Initial kernel scaffold; baseline (speedup 1.0000x reference)
import jax
import jax.numpy as jnp
from jax.experimental import pallas as pl


def kernel(indices, table, W, b):
    raise NotImplementedError("write your pallas kernel here")


# speedup vs baseline: 8.2628x; 8.2628x over previous
import functools

import jax
import jax.numpy as jnp
from jax.experimental import pallas as pl
from jax.experimental.pallas import tpu as pltpu
from jax.experimental.pallas import tpu_sc as plsc

N_PREFIX = 1000000
N_DIM_EMB = 32
N_ARY_OUT = 64

_GATHER_WINDOW = 128
_BM = 8192


def _sc_gather(table, idx_flat):
    m = idx_flat.shape[1]
    mesh = plsc.VectorSubcoreMesh(core_axis_name="core", subcore_axis_name="subcore")

    @functools.partial(
        pl.kernel,
        out_type=jax.ShapeDtypeStruct((m, N_DIM_EMB), jnp.float32),
        mesh=mesh,
        compiler_params=pltpu.CompilerParams(use_tc_tiling_on_sc=False),
    )
    def gather_kernel(table_hbm, idx_hbm, out_hbm):
        def body(i_vmem, o_vmem):
            pltpu.sync_copy(table_hbm.at[i_vmem.at[0]], o_vmem)

        pltpu.emit_pipeline(
            body,
            grid=(m // _GATHER_WINDOW,),
            in_specs=[pl.BlockSpec((1, _GATHER_WINDOW), lambda i: (0, i))],
            out_specs=[pl.BlockSpec((_GATHER_WINDOW, N_DIM_EMB), lambda i: (i, 0))],
            core_axis_name=("core", "subcore"),
            dimension_semantics=(pltpu.PARALLEL,),
        )(idx_hbm, out_hbm)

    return gather_kernel(table, idx_flat)


def _tc_project(h, W, b2d):
    m = h.shape[0]

    def matmul_body(h_ref, w_ref, b_ref, o_ref):
        o_ref[...] = (
            jnp.dot(h_ref[...], w_ref[...], preferred_element_type=jnp.float32)
            + b_ref[...]
        )

    return pl.pallas_call(
        matmul_body,
        grid=(m // _BM,),
        in_specs=[
            pl.BlockSpec((_BM, N_DIM_EMB), lambda i: (i, 0)),
            pl.BlockSpec((N_DIM_EMB, N_ARY_OUT), lambda i: (0, 0)),
            pl.BlockSpec((1, N_ARY_OUT), lambda i: (0, 0)),
        ],
        out_specs=pl.BlockSpec((_BM, N_ARY_OUT), lambda i: (i, 0)),
        out_shape=jax.ShapeDtypeStruct((m, N_ARY_OUT), jnp.float32),
    )(h, W, b2d)


def kernel(indices, table, W, b):
    batch, n_digits = indices.shape
    m = batch * n_digits
    idx_flat = indices.reshape(1, m)
    h = _sc_gather(table, idx_flat)
    logits = _tc_project(h, W, b.reshape(1, N_ARY_OUT))
    return logits.reshape(batch, n_digits, N_ARY_OUT)
